# Initial kernel scaffold; baseline (speedup 1.0000x reference)
import jax
import jax.numpy as jnp
from jax.experimental import pallas as pl


def kernel(xyz1, xyz2, points1, points2, W1, b1, g1, be1, W2, b2, g2, be2):
    raise NotImplementedError("write your pallas kernel here")


# speedup vs baseline: 18.1384x; 18.1384x over previous
import jax
import jax.numpy as jnp
from jax.experimental import pallas as pl

_NB1 = 256
_NB2 = 512
_NB3 = 1024


def _k1(x2t_ref, x1_ref, p2_ref, p1_ref, w1_ref, h1_ref, s_ref, ss_ref):
    b = pl.program_id(0)
    nt = pl.program_id(1)
    M = x2t_ref.shape[0]
    C2 = p2_ref.shape[0]
    nb1 = x1_ref.shape[1]

    d = jnp.zeros((M, nb1), jnp.float32)
    for c in range(3):
        diff = x2t_ref[:, c : c + 1] - x1_ref[c : c + 1, :]
        d = d + diff * diff

    sub_iota = jax.lax.broadcasted_iota(jnp.int32, (M, nb1), 0)

    def pick(dcur):
        m = jnp.min(dcur, axis=0, keepdims=True)
        i = jnp.min(
            jnp.where(dcur == m, sub_iota, M), axis=0, keepdims=True
        )
        dnext = jnp.where(sub_iota == i, jnp.float32(jnp.inf), dcur)
        return m, i, dnext

    m1, i1, dn = pick(d)
    m2, i2, dn = pick(dn)
    m3, i3, _ = pick(dn)

    r1 = 1.0 / jnp.maximum(m1, 1e-10)
    r2 = 1.0 / jnp.maximum(m2, 1e-10)
    r3 = 1.0 / jnp.maximum(m3, 1e-10)
    rs = r1 + r2 + r3
    st = (
        (r1 / rs) * (sub_iota == i1).astype(jnp.float32)
        + (r2 / rs) * (sub_iota == i2).astype(jnp.float32)
        + (r3 / rs) * (sub_iota == i3).astype(jnp.float32)
    )

    interp = jnp.dot(p2_ref[...], st, preferred_element_type=jnp.float32)
    h = jnp.dot(w1_ref[:, :C2], interp, preferred_element_type=jnp.float32)
    h = h + jnp.dot(w1_ref[:, C2:], p1_ref[...], preferred_element_type=jnp.float32)
    h1_ref[...] = h

    @pl.when(jnp.logical_and(b == 0, nt == 0))
    def _init():
        s_ref[...] = jnp.zeros_like(s_ref)
        ss_ref[...] = jnp.zeros_like(ss_ref)

    hh = h * h
    nb = h.shape[1]
    s_ref[...] += sum(h[:, j * 128 : (j + 1) * 128] for j in range(nb // 128))
    ss_ref[...] += sum(hh[:, j * 128 : (j + 1) * 128] for j in range(nb // 128))


def _k2(h1_ref, a_ref, c_ref, w2_ref, h2_ref, s_ref, ss_ref):
    b = pl.program_id(0)
    nt = pl.program_id(1)
    hn = jnp.maximum(a_ref[...] * h1_ref[...] + c_ref[...], 0.0)
    h2 = jnp.dot(w2_ref[...], hn, preferred_element_type=jnp.float32)
    h2_ref[...] = h2

    @pl.when(jnp.logical_and(b == 0, nt == 0))
    def _init():
        s_ref[...] = jnp.zeros_like(s_ref)
        ss_ref[...] = jnp.zeros_like(ss_ref)

    hh = h2 * h2
    nb = h2.shape[1]
    s_ref[...] += sum(h2[:, j * 128 : (j + 1) * 128] for j in range(nb // 128))
    ss_ref[...] += sum(hh[:, j * 128 : (j + 1) * 128] for j in range(nb // 128))


def _k3(h2_ref, a_ref, c_ref, o_ref):
    o_ref[...] = jnp.maximum(a_ref[...] * h2_ref[...] + c_ref[...], 0.0)


def _bn_coeffs(s, ss, cnt, g, be, eps=1e-5):
    mean = jnp.sum(s, axis=1) / cnt
    var = jnp.maximum(jnp.sum(ss, axis=1) / cnt - mean * mean, 0.0)
    a = g / jnp.sqrt(var + eps)
    c = be - mean * a
    return a[:, None], c[:, None]


def kernel(xyz1, xyz2, points1, points2, W1, b1, g1, be1, W2, b2, g2, be2):
    B, _, N = xyz1.shape
    M = xyz2.shape[2]
    C1 = points1.shape[1]
    C2 = points2.shape[1]
    H1 = W1.shape[0]
    H2 = W2.shape[0]
    cnt = B * N
    nb1 = min(_NB1, N)
    nb2 = min(_NB2, N)
    nb3 = min(_NB3, N)

    x2t = jnp.transpose(xyz2, (0, 2, 1))

    h1, s1, ss1 = pl.pallas_call(
        _k1,
        grid=(B, N // nb1),
        in_specs=[
            pl.BlockSpec((None, M, 3), lambda b, n: (b, 0, 0)),
            pl.BlockSpec((None, 3, nb1), lambda b, n: (b, 0, n)),
            pl.BlockSpec((None, C2, M), lambda b, n: (b, 0, 0)),
            pl.BlockSpec((None, C1, nb1), lambda b, n: (b, 0, n)),
            pl.BlockSpec((H1, C2 + C1), lambda b, n: (0, 0)),
        ],
        out_specs=[
            pl.BlockSpec((None, H1, nb1), lambda b, n: (b, 0, n)),
            pl.BlockSpec((H1, 128), lambda b, n: (0, 0)),
            pl.BlockSpec((H1, 128), lambda b, n: (0, 0)),
        ],
        out_shape=[
            jax.ShapeDtypeStruct((B, H1, N), jnp.float32),
            jax.ShapeDtypeStruct((H1, 128), jnp.float32),
            jax.ShapeDtypeStruct((H1, 128), jnp.float32),
        ],
    )(x2t, xyz1, points2, points1, W1)

    a1, c1 = _bn_coeffs(s1, ss1, cnt, g1, be1)

    h2, s2, ss2 = pl.pallas_call(
        _k2,
        grid=(B, N // nb2),
        in_specs=[
            pl.BlockSpec((None, H1, nb2), lambda b, n: (b, 0, n)),
            pl.BlockSpec((H1, 1), lambda b, n: (0, 0)),
            pl.BlockSpec((H1, 1), lambda b, n: (0, 0)),
            pl.BlockSpec((H2, H1), lambda b, n: (0, 0)),
        ],
        out_specs=[
            pl.BlockSpec((None, H2, nb2), lambda b, n: (b, 0, n)),
            pl.BlockSpec((H2, 128), lambda b, n: (0, 0)),
            pl.BlockSpec((H2, 128), lambda b, n: (0, 0)),
        ],
        out_shape=[
            jax.ShapeDtypeStruct((B, H2, N), jnp.float32),
            jax.ShapeDtypeStruct((H2, 128), jnp.float32),
            jax.ShapeDtypeStruct((H2, 128), jnp.float32),
        ],
    )(h1, a1, c1, W2)

    a2, c2 = _bn_coeffs(s2, ss2, cnt, g2, be2)

    out = pl.pallas_call(
        _k3,
        grid=(B, N // nb3),
        in_specs=[
            pl.BlockSpec((None, H2, nb3), lambda b, n: (b, 0, n)),
            pl.BlockSpec((H2, 1), lambda b, n: (0, 0)),
            pl.BlockSpec((H2, 1), lambda b, n: (0, 0)),
        ],
        out_specs=pl.BlockSpec((None, H2, nb3), lambda b, n: (b, 0, n)),
        out_shape=jax.ShapeDtypeStruct((B, H2, N), jnp.float32),
    )(h2, a2, c2)

    return out
